# baseline (speedup 1.0000x reference)
import functools

import jax
import jax.numpy as jnp
from jax import lax
from jax.experimental import pallas as pl
from jax.experimental.pallas import tpu as pltpu

TOPK = 2
BLK = 256
HT = 8


def _moe_body(m_ref, x_ref, w1_ref, b1_ref, w2_ref, b2_ref, w3_ref, b3_ref,
              gt_ref, tok_ref, y_ref, w1b, w2b, w3b, hb, acc, *, nb, htile):
    g = pl.program_id(0)
    t = pl.program_id(1)

    @pl.when(jnp.logical_and(g == 0, t == 0))
    def _():
        y_ref[...] = jnp.zeros_like(y_ref)

    @pl.when(g < m_ref[0, nb])
    def _():
        @pl.when(m_ref[1, g] == 1)
        def _():
            w2b[:, pl.ds(t * htile, htile)] = w2_ref[0].astype(jnp.bfloat16)
            w3b[pl.ds(t * htile, htile), :] = w3_ref[0].astype(jnp.bfloat16)

            @pl.when(t == 0)
            def _():
                w1b[...] = w1_ref[0].astype(jnp.bfloat16)

        @pl.when(t == 0)
        def _():
            h = jnp.dot(x_ref[...], w1b[...],
                        preferred_element_type=jnp.float32) + b1_ref[0]
            h = h * 0.5 * (1.0 + lax.erf(h * 0.7071067811865476))
            hb[...] = h.astype(jnp.bfloat16)

        h2 = jnp.dot(hb[...], w2b[:, pl.ds(t * htile, htile)],
                     preferred_element_type=jnp.float32) + b2_ref[0]
        part = jnp.dot(h2.astype(jnp.bfloat16), w3b[pl.ds(t * htile, htile), :],
                       preferred_element_type=jnp.float32)

        @pl.when(t == 0)
        def _():
            acc[...] = (part + b3_ref[0]) * gt_ref[...]

        @pl.when(t != 0)
        def _():
            acc[...] += part * gt_ref[...]

        @pl.when(t == HT - 1)
        def _():
            n_tok = y_ref.shape[0]
            rows = lax.broadcasted_iota(jnp.int32, (n_tok, BLK), 0)
            pt = (rows == jnp.broadcast_to(tok_ref[0], (n_tok, BLK)))
            y_ref[...] += jnp.dot(pt.astype(jnp.bfloat16),
                                  acc[...].astype(jnp.bfloat16),
                                  preferred_element_type=jnp.float32)


def kernel(z_pred, expert_eligibility, W1, b1, W2, b2, W3, b3):
    n, d = z_pred.shape
    e = W1.shape[0]
    h_dim = W1.shape[2]
    htile = h_dim // HT
    np_ = n * TOPK
    nb = np_ // BLK + e - 1
    npad = nb * BLK

    vals, idx = lax.top_k(expert_eligibility, TOPK)
    gates = jax.nn.softmax(vals, axis=-1)
    e_flat = idx.reshape(-1).astype(jnp.int32)
    g_flat = gates.reshape(-1)

    order = jnp.argsort(e_flat)
    e_sorted = e_flat[order]
    offs = jnp.searchsorted(e_sorted, jnp.arange(e, dtype=jnp.int32),
                            side="left").astype(jnp.int32)
    counts = jnp.diff(jnp.concatenate(
        [offs, jnp.array([np_], jnp.int32)])).astype(jnp.int32)
    nblk_e = (counts + BLK - 1) // BLK
    cum_blocks = jnp.cumsum(nblk_e)
    total_blocks = cum_blocks[-1]
    poffs = jnp.concatenate([jnp.zeros(1, jnp.int32),
                             jnp.cumsum(nblk_e * BLK)])[:e]

    bids = jnp.arange(nb, dtype=jnp.int32)
    block_expert = jnp.searchsorted(
        cum_blocks, jnp.minimum(bids, total_blocks - 1), side="right"
    ).astype(jnp.int32)
    changed = jnp.concatenate([jnp.ones(1, jnp.int32),
                               (jnp.diff(block_expert) != 0).astype(jnp.int32)])
    meta = jnp.stack([jnp.concatenate([block_expert, total_blocks[None]]),
                      jnp.concatenate([changed, jnp.zeros(1, jnp.int32)])])

    q = jnp.arange(npad, dtype=jnp.int32)
    eq = block_expert[q // BLK]
    r = q - poffs[eq]
    valid = r < counts[eq]
    src_pair = order[offs[eq] + jnp.minimum(r, counts[eq] - 1)]
    tok_src = (src_pair // TOPK).astype(jnp.int32)
    gate_col = jnp.where(valid, g_flat[src_pair], 0.0).reshape(npad, 1)
    tok_row = tok_src.reshape(nb, 1, BLK)
    x_sorted = z_pred.astype(jnp.bfloat16)[tok_src]

    def w2_map(g, t, m):
        return (m[0, g], 0, jnp.where(m[1, g] == 1, t, HT - 1))

    def w3_map(g, t, m):
        return (m[0, g], jnp.where(m[1, g] == 1, t, HT - 1), 0)

    grid_spec = pltpu.PrefetchScalarGridSpec(
        num_scalar_prefetch=1,
        grid=(nb, HT),
        in_specs=[
            pl.BlockSpec((BLK, d), lambda g, t, m: (g, 0)),
            pl.BlockSpec((1, d, h_dim), lambda g, t, m: (m[0, g], 0, 0)),
            pl.BlockSpec((1, 1, h_dim), lambda g, t, m: (m[0, g], 0, 0)),
            pl.BlockSpec((1, h_dim, htile), w2_map),
            pl.BlockSpec((1, 1, htile), lambda g, t, m: (m[0, g], 0, t)),
            pl.BlockSpec((1, htile, d), w3_map),
            pl.BlockSpec((1, 1, d), lambda g, t, m: (m[0, g], 0, 0)),
            pl.BlockSpec((BLK, 1), lambda g, t, m: (g, 0)),
            pl.BlockSpec((1, 1, BLK), lambda g, t, m: (g, 0, 0)),
        ],
        out_specs=pl.BlockSpec((n, d), lambda g, t, m: (0, 0)),
        scratch_shapes=[
            pltpu.VMEM((d, h_dim), jnp.bfloat16),
            pltpu.VMEM((h_dim, h_dim), jnp.bfloat16),
            pltpu.VMEM((h_dim, d), jnp.bfloat16),
            pltpu.VMEM((BLK, h_dim), jnp.bfloat16),
            pltpu.VMEM((BLK, d), jnp.float32),
        ],
    )
    y = pl.pallas_call(
        functools.partial(_moe_body, nb=nb, htile=htile),
        grid_spec=grid_spec,
        out_shape=jax.ShapeDtypeStruct((n, d), jnp.float32),
        compiler_params=pltpu.CompilerParams(
            dimension_semantics=("arbitrary", "arbitrary")),
    )(meta, x_sorted, W1, b1.reshape(e, 1, h_dim), W2,
      b2.reshape(e, 1, h_dim), W3, b3.reshape(e, 1, d), gate_col, tok_row)
    return y


# speedup vs baseline: 1.3375x; 1.3375x over previous
import functools

import jax
import jax.numpy as jnp
from jax import lax
from jax.experimental import pallas as pl
from jax.experimental.pallas import tpu as pltpu

TOPK = 2
BLK = 256
HT = 4


def _moe_body(m_ref, x_ref, w1_ref, b1_ref, w2_ref, b2_ref, w3_ref, b3_ref,
              gt_ref, out_ref, w1b, w2b, w3b, hb, *, nb, htile):
    g = pl.program_id(0)
    t = pl.program_id(1)

    @pl.when(g < m_ref[0, nb])
    def _():
        @pl.when(m_ref[1, g] == 1)
        def _():
            w2b[:, pl.ds(t * htile, htile)] = w2_ref[0].astype(jnp.bfloat16)
            w3b[pl.ds(t * htile, htile), :] = w3_ref[0].astype(jnp.bfloat16)

            @pl.when(t == 0)
            def _():
                w1b[...] = w1_ref[0].astype(jnp.bfloat16)

        @pl.when(t == 0)
        def _():
            h = jnp.dot(x_ref[...].astype(jnp.bfloat16), w1b[...],
                        preferred_element_type=jnp.float32) + b1_ref[0]
            h = h * 0.5 * (1.0 + lax.erf(h * 0.7071067811865476))
            hb[...] = h.astype(jnp.bfloat16)

        h2 = jnp.dot(hb[...], w2b[:, pl.ds(t * htile, htile)],
                     preferred_element_type=jnp.float32) + b2_ref[0]
        part = jnp.dot(h2.astype(jnp.bfloat16), w3b[pl.ds(t * htile, htile), :],
                       preferred_element_type=jnp.float32)

        @pl.when(t == 0)
        def _():
            out_ref[...] = (part + b3_ref[0]) * gt_ref[...]

        @pl.when(t != 0)
        def _():
            out_ref[...] += part * gt_ref[...]


def kernel(z_pred, expert_eligibility, W1, b1, W2, b2, W3, b3):
    n, d = z_pred.shape
    e = W1.shape[0]
    h_dim = W1.shape[2]
    htile = h_dim // HT
    np_ = n * TOPK
    nb = np_ // BLK + e - 1
    npad = nb * BLK

    vals, idx = lax.top_k(expert_eligibility, TOPK)
    gates = jax.nn.softmax(vals, axis=-1)
    e_flat = idx.reshape(-1).astype(jnp.int32)
    g_flat = gates.reshape(-1)

    onehot = (e_flat[:, None] == jnp.arange(e, dtype=jnp.int32)[None, :])
    csum = jnp.cumsum(onehot.astype(jnp.int32), axis=0)
    rank = jnp.take_along_axis(csum, e_flat[:, None], axis=1)[:, 0] - 1
    counts = csum[-1]
    nblk_e = (counts + BLK - 1) // BLK
    cum_blocks = jnp.cumsum(nblk_e)
    total_blocks = cum_blocks[-1]
    poffs = jnp.concatenate([jnp.zeros(1, jnp.int32),
                             jnp.cumsum(nblk_e * BLK)])[:e]
    dest = poffs[e_flat] + rank

    bids = jnp.arange(nb, dtype=jnp.int32)
    block_expert = jnp.searchsorted(
        cum_blocks, jnp.minimum(bids, total_blocks - 1), side="right"
    ).astype(jnp.int32)
    changed = jnp.concatenate([jnp.ones(1, jnp.int32),
                               (jnp.diff(block_expert) != 0).astype(jnp.int32)])
    meta = jnp.stack([jnp.concatenate([block_expert, total_blocks[None]]),
                      jnp.concatenate([changed, jnp.zeros(1, jnp.int32)])])

    tok_src = jnp.zeros(npad, jnp.int32).at[dest].set(
        jnp.arange(np_, dtype=jnp.int32) // TOPK)
    gate_col = jnp.zeros((npad,), jnp.float32).at[dest].set(
        g_flat).reshape(npad, 1)
    x_sorted = z_pred[tok_src]

    def w2_map(g, t, m):
        return (m[0, g], 0, jnp.where(m[1, g] == 1, t, HT - 1))

    def w3_map(g, t, m):
        return (m[0, g], jnp.where(m[1, g] == 1, t, HT - 1), 0)

    grid_spec = pltpu.PrefetchScalarGridSpec(
        num_scalar_prefetch=1,
        grid=(nb, HT),
        in_specs=[
            pl.BlockSpec((BLK, d), lambda g, t, m: (g, 0)),
            pl.BlockSpec((1, d, h_dim), lambda g, t, m: (m[0, g], 0, 0)),
            pl.BlockSpec((1, 1, h_dim), lambda g, t, m: (m[0, g], 0, 0)),
            pl.BlockSpec((1, h_dim, htile), w2_map),
            pl.BlockSpec((1, 1, htile), lambda g, t, m: (m[0, g], 0, t)),
            pl.BlockSpec((1, htile, d), w3_map),
            pl.BlockSpec((1, 1, d), lambda g, t, m: (m[0, g], 0, 0)),
            pl.BlockSpec((BLK, 1), lambda g, t, m: (g, 0)),
        ],
        out_specs=pl.BlockSpec((BLK, d), lambda g, t, m: (g, 0)),
        scratch_shapes=[
            pltpu.VMEM((d, h_dim), jnp.bfloat16),
            pltpu.VMEM((h_dim, h_dim), jnp.bfloat16),
            pltpu.VMEM((h_dim, d), jnp.bfloat16),
            pltpu.VMEM((BLK, h_dim), jnp.bfloat16),
        ],
    )
    out_rows = pl.pallas_call(
        functools.partial(_moe_body, nb=nb, htile=htile),
        grid_spec=grid_spec,
        out_shape=jax.ShapeDtypeStruct((npad, d), jnp.float32),
        compiler_params=pltpu.CompilerParams(
            dimension_semantics=("arbitrary", "arbitrary")),
    )(meta, x_sorted, W1, b1.reshape(e, 1, h_dim), W2,
      b2.reshape(e, 1, h_dim), W3, b3.reshape(e, 1, d), gate_col)

    dr = dest.reshape(n, TOPK)
    y = out_rows[dr[:, 0]] + out_rows[dr[:, 1]]
    return y
